# baseline (speedup 1.0000x reference)
import functools

import jax
import jax.numpy as jnp
from jax import lax
from jax.experimental import pallas as pl
from jax.experimental.pallas import tpu as pltpu
from jax.experimental.pallas import tpu_sc as plsc

_B = 16384
_D = 64
_NC = 2
_NS = 16
_NW = _NC * _NS
_BPW = _B // _NW
_L = 16
_GPW = _BPW // _L


def _gather_body(idx_hbm, tab_hbm, out_hbm, idx_v, pair_v, rows_v, sem):
    wid = lax.axis_index("s") * _NC + lax.axis_index("c")
    base = wid * _BPW
    pltpu.sync_copy(idx_hbm.at[pl.ds(base, _BPW)], idx_v)

    def halve(g, carry):
        sl = pl.ds(g * _L, _L)
        pair_v[sl] = lax.shift_right_logical(idx_v[sl], 1)
        return carry

    lax.fori_loop(0, _GPW, halve, 0)
    pltpu.async_copy(tab_hbm.at[pair_v], rows_v, sem).wait()
    pltpu.sync_copy(rows_v, out_hbm.at[pl.ds(base, _BPW)])


_gather_sc = functools.partial(
    pl.kernel,
    mesh=plsc.VectorSubcoreMesh(core_axis_name="c", subcore_axis_name="s"),
    out_type=jax.ShapeDtypeStruct((_B, 2 * _D), jnp.float32),
    scratch_types=[
        pltpu.VMEM((_BPW,), jnp.int32),
        pltpu.VMEM((_BPW,), jnp.int32),
        pltpu.VMEM((_BPW, 2 * _D), jnp.float32),
        pltpu.SemaphoreType.DMA,
    ],
    compiler_params=pltpu.CompilerParams(needs_layout_passes=False),
)(_gather_body)


def _score_body(subj_hbm, obj_hbm, rel_hbm, ps_hbm, po_hbm, pr_hbm, out_hbm,
                idx_s, idx_o, idx_r, rows_s, rows_o, rows_r, out_v,
                sem_s, sem_o, sem_r):
    wid = lax.axis_index("s") * _NC + lax.axis_index("c")
    base = wid * _BPW

    pltpu.sync_copy(subj_hbm.at[pl.ds(base, _BPW)], idx_s)
    pltpu.sync_copy(obj_hbm.at[pl.ds(base, _BPW)], idx_o)
    pltpu.sync_copy(rel_hbm.at[pl.ds(base, _BPW)], idx_r)
    lane = lax.iota(jnp.int32, _L)

    for c in range(2):
        half = _BPW // 2
        hsl = pl.ds(base + c * half, half)
        cs = pltpu.async_copy(ps_hbm.at[hsl], rows_s, sem_s)
        co = pltpu.async_copy(po_hbm.at[hsl], rows_o, sem_o)
        cr = pltpu.async_copy(pr_hbm.at[hsl], rows_r, sem_r)
        cs.wait()
        co.wait()
        cr.wait()

        def group(g, carry):
            row = g * _L + lane
            off = c * half + g * _L
            col_s = (idx_s[pl.ds(off, _L)] & 1) * _D
            col_o = (idx_o[pl.ds(off, _L)] & 1) * _D
            col_r = (idx_r[pl.ds(off, _L)] & 1) * _D
            acc = jnp.zeros((_L,), jnp.float32)
            for d in range(_D):
                s = plsc.load_gather(rows_s, [row, col_s + d])
                o = plsc.load_gather(rows_o, [row, col_o + d])
                r = plsc.load_gather(rows_r, [row, col_r + d])
                acc = acc + s * r * o
            out_v[pl.ds(off, _L)] = acc
            return carry

        lax.fori_loop(0, _GPW // 2, group, 0)

    pltpu.sync_copy(out_v, out_hbm.at[pl.ds(base, _BPW)])


_score_sc = functools.partial(
    pl.kernel,
    mesh=plsc.VectorSubcoreMesh(core_axis_name="c", subcore_axis_name="s"),
    out_type=jax.ShapeDtypeStruct((_B,), jnp.float32),
    scratch_types=[
        pltpu.VMEM((_BPW,), jnp.int32),
        pltpu.VMEM((_BPW,), jnp.int32),
        pltpu.VMEM((_BPW,), jnp.int32),
        pltpu.VMEM((_BPW // 2, 2 * _D), jnp.float32),
        pltpu.VMEM((_BPW // 2, 2 * _D), jnp.float32),
        pltpu.VMEM((_BPW // 2, 2 * _D), jnp.float32),
        pltpu.VMEM((_BPW,), jnp.float32),
        pltpu.SemaphoreType.DMA,
        pltpu.SemaphoreType.DMA,
        pltpu.SemaphoreType.DMA,
    ],
    compiler_params=pltpu.CompilerParams(needs_layout_passes=False),
)(_score_body)


def kernel(triples, entity_table, relation_table):
    t = triples.astype(jnp.int32)
    s_idx, o_idx, r_idx = t[:, 0], t[:, 1], t[:, 2]
    ent2 = entity_table.reshape(500000, 2 * _D)
    rel2 = relation_table.reshape(500000, 2 * _D)
    ps = _gather_sc(s_idx, ent2)
    po = _gather_sc(o_idx, ent2)
    pr = _gather_sc(r_idx, rel2)
    scores = _score_sc(s_idx, o_idx, r_idx, ps, po, pr)
    return scores.reshape(_B, 1)


# speedup vs baseline: 1.5460x; 1.5460x over previous
import functools

import jax
import jax.numpy as jnp
from jax import lax
from jax.experimental import pallas as pl
from jax.experimental.pallas import tpu as pltpu
from jax.experimental.pallas import tpu_sc as plsc

_B = 16384
_D = 64
_N = 1000000
_HALF = 524288
_W = 1024
_NBLK = _HALF // _W
_LASTBLK = (_N - 1) // _W
_NC = 2
_NS = 16
_NW = _NC * _NS
_BPW = _B // _NW
_L = 16
_CH = 256
_NCH = _BPW // _CH
_GPC = _CH // _L


def _relayout_body(e1, e2, r1, r2, out_e, out_r):
    out_e[...] = jnp.concatenate([e1[...].T, e2[...].T], axis=1)
    out_r[...] = jnp.concatenate([r1[...].T, r2[...].T], axis=1)


_relayout = pl.pallas_call(
    _relayout_body,
    grid=(_NBLK,),
    in_specs=[
        pl.BlockSpec((_D, _W), lambda j: (0, j)),
        pl.BlockSpec((_D, _W), lambda j: (0, jnp.minimum(j + _NBLK, _LASTBLK))),
        pl.BlockSpec((_D, _W), lambda j: (0, j)),
        pl.BlockSpec((_D, _W), lambda j: (0, jnp.minimum(j + _NBLK, _LASTBLK))),
    ],
    out_specs=[
        pl.BlockSpec((_W, 2 * _D), lambda j: (j, 0)),
        pl.BlockSpec((_W, 2 * _D), lambda j: (j, 0)),
    ],
    out_shape=[
        jax.ShapeDtypeStruct((_HALF, 2 * _D), jnp.float32),
        jax.ShapeDtypeStruct((_HALF, 2 * _D), jnp.float32),
    ],
    compiler_params=pltpu.CompilerParams(
        dimension_semantics=("arbitrary",)),
)


def _distmult_body(subj_hbm, obj_hbm, rel_hbm, ent_hbm, relt_hbm, out_hbm,
                   idx_s, idx_o, idx_r, row_s, row_o, row_r,
                   rows_s, rows_o, rows_r, out_v,
                   sem_s, sem_o, sem_r):
    wid = lax.axis_index("s") * _NC + lax.axis_index("c")
    base = wid * _BPW

    pltpu.sync_copy(subj_hbm.at[pl.ds(base, _BPW)], idx_s)
    pltpu.sync_copy(obj_hbm.at[pl.ds(base, _BPW)], idx_o)
    pltpu.sync_copy(rel_hbm.at[pl.ds(base, _BPW)], idx_r)

    def fold(g, carry):
        sl = pl.ds(g * _L, _L)
        row_s[sl] = idx_s[sl] & (_HALF - 1)
        row_o[sl] = idx_o[sl] & (_HALF - 1)
        row_r[sl] = idx_r[sl] & (_HALF - 1)
        return carry

    lax.fori_loop(0, _BPW // _L, fold, 0)

    lane = lax.iota(jnp.int32, _L)

    for c in range(_NCH):
        csl = pl.ds(c * _CH, _CH)
        cs = pltpu.async_copy(ent_hbm.at[row_s.at[csl]], rows_s, sem_s)
        co = pltpu.async_copy(ent_hbm.at[row_o.at[csl]], rows_o, sem_o)
        cr = pltpu.async_copy(relt_hbm.at[row_r.at[csl]], rows_r, sem_r)
        cs.wait()
        co.wait()
        cr.wait()

        def group(g, carry):
            row = g * _L + lane
            off = c * _CH + g * _L
            col_s = lax.shift_right_logical(idx_s[pl.ds(off, _L)], 19) * _D
            col_o = lax.shift_right_logical(idx_o[pl.ds(off, _L)], 19) * _D
            col_r = lax.shift_right_logical(idx_r[pl.ds(off, _L)], 19) * _D
            acc = jnp.zeros((_L,), jnp.float32)
            for d in range(_D):
                s = plsc.load_gather(rows_s, [row, col_s + d])
                o = plsc.load_gather(rows_o, [row, col_o + d])
                r = plsc.load_gather(rows_r, [row, col_r + d])
                acc = acc + s * r * o
            out_v[pl.ds(off, _L)] = acc
            return carry

        lax.fori_loop(0, _GPC, group, 0)

    pltpu.sync_copy(out_v, out_hbm.at[pl.ds(base, _BPW)])


_distmult_sc = functools.partial(
    pl.kernel,
    mesh=plsc.VectorSubcoreMesh(core_axis_name="c", subcore_axis_name="s"),
    out_type=jax.ShapeDtypeStruct((_B,), jnp.float32),
    scratch_types=[
        pltpu.VMEM((_BPW,), jnp.int32),
        pltpu.VMEM((_BPW,), jnp.int32),
        pltpu.VMEM((_BPW,), jnp.int32),
        pltpu.VMEM((_BPW,), jnp.int32),
        pltpu.VMEM((_BPW,), jnp.int32),
        pltpu.VMEM((_BPW,), jnp.int32),
        pltpu.VMEM((_CH, 2 * _D), jnp.float32),
        pltpu.VMEM((_CH, 2 * _D), jnp.float32),
        pltpu.VMEM((_CH, 2 * _D), jnp.float32),
        pltpu.VMEM((_BPW,), jnp.float32),
        pltpu.SemaphoreType.DMA,
        pltpu.SemaphoreType.DMA,
        pltpu.SemaphoreType.DMA,
    ],
    compiler_params=pltpu.CompilerParams(needs_layout_passes=False),
)(_distmult_body)


def kernel(triples, entity_table, relation_table):
    t = triples.astype(jnp.int32)
    ent_t, rel_t = entity_table.T, relation_table.T
    ent_r, rel_r = _relayout(ent_t, ent_t, rel_t, rel_t)
    scores = _distmult_sc(t[:, 0], t[:, 1], t[:, 2], ent_r, rel_r)
    return scores.reshape(_B, 1)
